# baseline (speedup 1.0000x reference)
import functools

import jax
import jax.numpy as jnp
from jax import lax
from jax.experimental import pallas as pl
from jax.experimental.pallas import tpu as pltpu
from jax.experimental.pallas import tpu_sc as plsc

_IMG = 96
_RADIUS = 0.05
_K = 8
_CH = 256
_NSUB = 32
_BIG = 2 ** 30


def _transform_kernel(params_ref, pts_ref, out_ref):
    pts = pts_ref[0]
    x = pts[0:1, :]
    y = pts[1:2, :]
    z = pts[2:3, :]
    (r00, r01, r02, r10, r11, r12, r20, r21, r22, t0, t1, t2, foc) = [
        params_ref[0, 0, j] for j in range(13)]

    def _b(v):
        return v.astype(jnp.bfloat16).astype(jnp.float32)

    x, y, z = _b(x), _b(y), _b(z)
    r00, r01, r02 = _b(r00), _b(r01), _b(r02)
    r10, r11, r12 = _b(r10), _b(r11), _b(r12)
    r20, r21, r22 = _b(r20), _b(r21), _b(r22)
    xv = x * r00 + y * r10 + z * r20 + t0
    yv = x * r01 + y * r11 + z * r21 + t1
    zv = x * r02 + y * r12 + z * r22 + t2
    z_safe = jnp.where(jnp.abs(zv) < 1e-5, 1e-5, zv)
    px = foc * xv / z_safe
    py = foc * yv / z_safe
    out_ref[0] = jnp.concatenate([px, py, zv], axis=0)


def _sc_bin_kernel(pxyz_hbm, cand_hbm, counts_hbm, pts_v, rpx_v, rdy_v, rpz_v,
                   rid_v, cnt_v, *, n, rows_per):
    wid = lax.axis_index("s") * 2 + lax.axis_index("c")
    row0 = wid * rows_per
    b = row0 // _IMG
    pltpu.sync_copy(pxyz_hbm.at[b], pts_v)
    lane = lax.broadcasted_iota(jnp.int32, (16,), 0)
    counts_reg = jnp.zeros((16,), jnp.int32)
    r2 = _RADIUS * _RADIUS
    for j in range(rows_per):
        flat = row0 + j
        r = flat - b * _IMG
        rf = jnp.full((16,), r, jnp.int32).astype(jnp.float32)
        y_r = 1.0 - 2.0 * (rf + 0.5) / jnp.full((16,), float(_IMG), jnp.float32)

        def chunk_body(c, cnt):
            base = c * 16
            pxv = pts_v[0, pl.ds(base, 16)]
            pyv = pts_v[1, pl.ds(base, 16)]
            pzv = pts_v[2, pl.ds(base, 16)]
            dy = pyv - y_r
            dy2 = dy * dy
            m = (dy2 < r2) & (pzv > 0.0)
            idxf = (lane + base).astype(jnp.float32)
            mi = m.astype(jnp.int32)
            csum = plsc.cumsum(mi)
            pos = jnp.where(m, cnt + csum - 1, n + lane)
            plsc.store_scatter(rpx_v, [pos], pxv)
            plsc.store_scatter(rdy_v, [pos], dy2)
            plsc.store_scatter(rpz_v, [pos], pzv)
            plsc.store_scatter(rid_v, [pos], idxf)
            return cnt + jnp.sum(mi)

        cnt = lax.fori_loop(0, n // 16, chunk_body, jnp.int32(0))
        pltpu.sync_copy(rpx_v.at[pl.ds(0, n)], cand_hbm.at[flat, 0])
        pltpu.sync_copy(rdy_v.at[pl.ds(0, n)], cand_hbm.at[flat, 1])
        pltpu.sync_copy(rpz_v.at[pl.ds(0, n)], cand_hbm.at[flat, 2])
        pltpu.sync_copy(rid_v.at[pl.ds(0, n)], cand_hbm.at[flat, 3])
        counts_reg = jnp.where(lane == j, jnp.full((16,), cnt, jnp.int32),
                               counts_reg)
    cnt_v[...] = counts_reg
    pltpu.sync_copy(cnt_v, counts_hbm.at[wid])


_BIGF = float(2 ** 30)


def _select8(z, ci):
    zs, cs = [], []
    for kk in range(_K):
        mv = jnp.min(z, axis=1, keepdims=True)
        live = mv < jnp.inf
        pos = jnp.min(jnp.where(z == mv, ci, _BIGF), axis=1, keepdims=True)
        pos = jnp.where(live, pos, _BIGF)
        if kk + 1 < _K:
            z = jnp.where(ci == pos, jnp.inf, z)
        zs.append(mv)
        cs.append(pos)
    return zs, cs


def _topk_kernel(counts_ref, cand_ref, idx_ref, zbuf_ref, *, n):
    i = pl.program_id(0)
    count = counts_ref[i]
    nch = lax.div(count + (_CH - 1), _CH)
    r2 = _RADIUS * _RADIUS
    xs = 1.0 - 2.0 * (
        lax.broadcasted_iota(jnp.int32, (_IMG, 1), 0).astype(jnp.float32)
        + 0.5) / _IMG
    iota_c = lax.broadcasted_iota(jnp.int32, (_IMG, _CH), 1)

    state = (jnp.full((_IMG, _K), jnp.inf, jnp.float32),
             jnp.full((_IMG, _K), _BIGF, jnp.float32))

    def chunk(jc, st):
        zs, cs = st
        base = pl.multiple_of(jc * _CH, _CH)
        px = cand_ref[0, 0:1, pl.ds(base, _CH)]
        dy2 = cand_ref[0, 1:2, pl.ds(base, _CH)]
        pz = cand_ref[0, 2:3, pl.ds(base, _CH)]
        ci = cand_ref[0, 3:4, pl.ds(base, _CH)]
        d2 = (px - xs) ** 2 + dy2
        gpos = iota_c + base
        valid = (d2 < r2) & (pz > 0.0) & (gpos < count)
        zm = jnp.where(valid, pz, jnp.inf)
        ciq = jnp.where(valid, ci, _BIGF)
        czs, ccs = _select8(zm, ciq)
        z16 = jnp.concatenate([zs] + czs, axis=1)
        c16 = jnp.concatenate([cs] + ccs, axis=1)
        nzs, ncs = _select8(z16, c16)
        return (jnp.concatenate(nzs, axis=1), jnp.concatenate(ncs, axis=1))

    zs, cs = lax.fori_loop(0, nch, chunk, state)
    fin = zs < jnp.inf
    idx_ref[0, 0] = jnp.where(fin, cs.astype(jnp.int32), -1)
    zbuf_ref[0, 0] = jnp.where(fin, zs, -1.0)


def _sc_dist_kernel(pxyz_hbm, idx_hbm, dist_hbm, pts_v, idx_v, dst_v, *,
                    n, rows_per):
    wid = lax.axis_index("s") * 2 + lax.axis_index("c")
    row0 = wid * rows_per
    b = row0 // _IMG
    pltpu.sync_copy(pxyz_hbm.at[b], pts_v)
    lane = lax.broadcasted_iota(jnp.int32, (16,), 0)
    zero16 = jnp.zeros((16,), jnp.int32)
    one16 = jnp.full((16,), 1, jnp.int32)
    img_f = jnp.full((16,), float(_IMG), jnp.float32)
    rowlen = _IMG * _K
    for j in range(rows_per):
        flat = row0 + j
        r = flat - b * _IMG
        rf = jnp.full((16,), r, jnp.int32).astype(jnp.float32)
        ys = 1.0 - 2.0 * (rf + 0.5) / img_f
        pltpu.sync_copy(idx_hbm.at[flat], idx_v)

        def chunk_body(c, carry):
            base = c * 16
            iv = idx_v[pl.ds(base, 16)]
            icl = jnp.maximum(iv, zero16)
            pxg = plsc.load_gather(pts_v, [zero16, icl])
            pyg = plsc.load_gather(pts_v, [one16, icl])
            wcol = lax.shift_right_logical(lane + base, 3)
            xsv = 1.0 - 2.0 * (wcol.astype(jnp.float32) + 0.5) / img_f
            dx = pxg - xsv
            dyv = pyg - ys
            d = dx * dx + dyv * dyv
            dst_v[pl.ds(base, 16)] = jnp.where(iv < 0, -1.0, d)
            return carry

        lax.fori_loop(0, rowlen // 16, chunk_body, jnp.int32(0))
        pltpu.sync_copy(dst_v, dist_hbm.at[flat])


def kernel(points, R, T, focal):
    B, N, _ = points.shape
    img, k = _IMG, _K
    nrows = B * img
    rows_per = nrows // _NSUB
    pts = jnp.transpose(points, (0, 2, 1))
    params = jnp.concatenate(
        [R.reshape(B, 9), T.reshape(B, 3), focal.reshape(B, 1),
         jnp.zeros((B, 3), jnp.float32)], axis=1).reshape(B, 1, 16)

    pxyz = pl.pallas_call(
        _transform_kernel,
        out_shape=jax.ShapeDtypeStruct((B, 3, N), jnp.float32),
        grid=(B,),
        in_specs=[
            pl.BlockSpec((1, 1, 16), lambda b: (b, 0, 0),
                         memory_space=pltpu.SMEM),
            pl.BlockSpec((1, 3, N), lambda b: (b, 0, 0)),
        ],
        out_specs=pl.BlockSpec((1, 3, N), lambda b: (b, 0, 0)),
    )(params, pts)

    mesh = plsc.VectorSubcoreMesh(core_axis_name="c", subcore_axis_name="s")
    cand, counts2d = pl.kernel(
        functools.partial(_sc_bin_kernel, n=N, rows_per=rows_per),
        out_type=[jax.ShapeDtypeStruct((nrows, 4, N), jnp.float32),
                  jax.ShapeDtypeStruct((_NSUB, 16), jnp.int32)],
        mesh=mesh,
        compiler_params=pltpu.CompilerParams(needs_layout_passes=False),
        scratch_types=[pltpu.VMEM((3, N), jnp.float32),
                       pltpu.VMEM((N + 16,), jnp.float32),
                       pltpu.VMEM((N + 16,), jnp.float32),
                       pltpu.VMEM((N + 16,), jnp.float32),
                       pltpu.VMEM((N + 16,), jnp.float32),
                       pltpu.VMEM((16,), jnp.int32)],
    )(pxyz)

    counts_flat = counts2d[:, :rows_per].reshape(nrows)

    out_shapes = (
        jax.ShapeDtypeStruct((B, img, img, k), jnp.int32),
        jax.ShapeDtypeStruct((B, img, img, k), jnp.float32),
    )
    out_spec = pl.BlockSpec((1, 1, img, k),
                            lambda i, counts: (i // img, i % img, 0, 0))
    idx, zbuf = pl.pallas_call(
        functools.partial(_topk_kernel, n=N),
        out_shape=out_shapes,
        grid_spec=pltpu.PrefetchScalarGridSpec(
            num_scalar_prefetch=1,
            grid=(nrows,),
            in_specs=[pl.BlockSpec((1, 4, N), lambda i, counts: (i, 0, 0))],
            out_specs=(out_spec, out_spec),
        ),
    )(counts_flat, cand)

    rowlen = img * k
    dists = pl.kernel(
        functools.partial(_sc_dist_kernel, n=N, rows_per=rows_per),
        out_type=jax.ShapeDtypeStruct((nrows, rowlen), jnp.float32),
        mesh=mesh,
        compiler_params=pltpu.CompilerParams(needs_layout_passes=False),
        scratch_types=[pltpu.VMEM((3, N), jnp.float32),
                       pltpu.VMEM((rowlen,), jnp.int32),
                       pltpu.VMEM((rowlen,), jnp.float32)],
    )(pxyz, idx.reshape(nrows, rowlen))

    return idx, zbuf, dists.reshape(B, img, img, k)


# speedup vs baseline: 1.5976x; 1.5976x over previous
import functools

import jax
import jax.numpy as jnp
from jax import lax
from jax.experimental import pallas as pl
from jax.experimental.pallas import tpu as pltpu
from jax.experimental.pallas import tpu_sc as plsc

_IMG = 96
_RADIUS = 0.05
_K = 8
_CH = 1024
_NSUB = 32
_BIG = 2 ** 30


def _transform_kernel(params_ref, pts_ref, out_ref):
    pts = pts_ref[0]
    x = pts[0:1, :]
    y = pts[1:2, :]
    z = pts[2:3, :]
    (r00, r01, r02, r10, r11, r12, r20, r21, r22, t0, t1, t2, foc) = [
        params_ref[0, 0, j] for j in range(13)]

    def _b(v):
        return v.astype(jnp.bfloat16).astype(jnp.float32)

    x, y, z = _b(x), _b(y), _b(z)
    r00, r01, r02 = _b(r00), _b(r01), _b(r02)
    r10, r11, r12 = _b(r10), _b(r11), _b(r12)
    r20, r21, r22 = _b(r20), _b(r21), _b(r22)
    xv = x * r00 + y * r10 + z * r20 + t0
    yv = x * r01 + y * r11 + z * r21 + t1
    zv = x * r02 + y * r12 + z * r22 + t2
    z_safe = jnp.where(jnp.abs(zv) < 1e-5, 1e-5, zv)
    px = foc * xv / z_safe
    py = foc * yv / z_safe
    out_ref[0] = jnp.concatenate([px, py, zv], axis=0)


def _sc_bin_kernel(pxyz_hbm, cand_hbm, counts_hbm, pts_v, rpx_v, rdy_v, rpz_v,
                   rid_v, cnt_v, *, n, rows_per):
    wid = lax.axis_index("s") * 2 + lax.axis_index("c")
    row0 = wid * rows_per
    b = row0 // _IMG
    pltpu.sync_copy(pxyz_hbm.at[b], pts_v)
    lane = lax.broadcasted_iota(jnp.int32, (16,), 0)
    counts_reg = jnp.zeros((16,), jnp.int32)
    r2 = _RADIUS * _RADIUS
    for j in range(rows_per):
        flat = row0 + j
        r = flat - b * _IMG
        rf = jnp.full((16,), r, jnp.int32).astype(jnp.float32)
        y_r = 1.0 - 2.0 * (rf + 0.5) / jnp.full((16,), float(_IMG), jnp.float32)

        def chunk_body(c, cnt):
            base = c * 16
            pxv = pts_v[0, pl.ds(base, 16)]
            pyv = pts_v[1, pl.ds(base, 16)]
            pzv = pts_v[2, pl.ds(base, 16)]
            dy = pyv - y_r
            dy2 = dy * dy
            m = (dy2 < r2) & (pzv > 0.0)
            idxf = (lane + base).astype(jnp.float32)
            mi = m.astype(jnp.int32)
            csum = plsc.cumsum(mi)
            pos = jnp.where(m, cnt + csum - 1, n + lane)
            plsc.store_scatter(rpx_v, [pos], pxv)
            plsc.store_scatter(rdy_v, [pos], dy2)
            plsc.store_scatter(rpz_v, [pos], pzv)
            plsc.store_scatter(rid_v, [pos], idxf)
            return cnt + jnp.sum(mi)

        cnt = lax.fori_loop(0, n // 16, chunk_body, jnp.int32(0))
        pltpu.sync_copy(rpx_v.at[pl.ds(0, n)], cand_hbm.at[flat, 0])
        pltpu.sync_copy(rdy_v.at[pl.ds(0, n)], cand_hbm.at[flat, 1])
        pltpu.sync_copy(rpz_v.at[pl.ds(0, n)], cand_hbm.at[flat, 2])
        pltpu.sync_copy(rid_v.at[pl.ds(0, n)], cand_hbm.at[flat, 3])
        counts_reg = jnp.where(lane == j, jnp.full((16,), cnt, jnp.int32),
                               counts_reg)
    cnt_v[...] = counts_reg
    pltpu.sync_copy(cnt_v, counts_hbm.at[wid])


_BIGF = float(2 ** 30)


def _select8(z, ci):
    zs, cs = [], []
    for kk in range(_K):
        mv = jnp.min(z, axis=1, keepdims=True)
        live = mv < jnp.inf
        pos = jnp.min(jnp.where(z == mv, ci, _BIGF), axis=1, keepdims=True)
        pos = jnp.where(live, pos, _BIGF)
        if kk + 1 < _K:
            z = jnp.where(ci == pos, jnp.inf, z)
        zs.append(mv)
        cs.append(pos)
    return zs, cs


def _topk_kernel(counts_ref, cand_ref, idx_ref, zbuf_ref, *, n):
    i = pl.program_id(0)
    count = counts_ref[i]
    nch = lax.div(count + (_CH - 1), _CH)
    r2 = _RADIUS * _RADIUS
    xs = 1.0 - 2.0 * (
        lax.broadcasted_iota(jnp.int32, (_IMG, 1), 0).astype(jnp.float32)
        + 0.5) / _IMG
    iota_c = lax.broadcasted_iota(jnp.int32, (_IMG, _CH), 1)

    state = (jnp.full((_IMG, _K), jnp.inf, jnp.float32),
             jnp.full((_IMG, _K), _BIGF, jnp.float32))

    def chunk(jc, st):
        zs, cs = st
        base = pl.multiple_of(jc * _CH, _CH)
        px = cand_ref[0, 0:1, pl.ds(base, _CH)]
        dy2 = cand_ref[0, 1:2, pl.ds(base, _CH)]
        pz = cand_ref[0, 2:3, pl.ds(base, _CH)]
        ci = cand_ref[0, 3:4, pl.ds(base, _CH)]
        d2 = (px - xs) ** 2 + dy2
        gpos = iota_c + base
        valid = (d2 < r2) & (pz > 0.0) & (gpos < count)
        zm = jnp.where(valid, pz, jnp.inf)
        ciq = jnp.where(valid, ci, _BIGF)
        czs, ccs = _select8(zm, ciq)
        z16 = jnp.concatenate([zs] + czs, axis=1)
        c16 = jnp.concatenate([cs] + ccs, axis=1)
        nzs, ncs = _select8(z16, c16)
        return (jnp.concatenate(nzs, axis=1), jnp.concatenate(ncs, axis=1))

    zs, cs = lax.fori_loop(0, nch, chunk, state)
    fin = zs < jnp.inf
    idx_ref[0, 0] = jnp.where(fin, cs.astype(jnp.int32), -1)
    zbuf_ref[0, 0] = jnp.where(fin, zs, -1.0)


def _sc_dist_kernel(pxyz_hbm, idx_hbm, dist_hbm, pts_v, idx_v, dst_v, *,
                    n, rows_per):
    wid = lax.axis_index("s") * 2 + lax.axis_index("c")
    row0 = wid * rows_per
    b = row0 // _IMG
    pltpu.sync_copy(pxyz_hbm.at[b], pts_v)
    lane = lax.broadcasted_iota(jnp.int32, (16,), 0)
    zero16 = jnp.zeros((16,), jnp.int32)
    one16 = jnp.full((16,), 1, jnp.int32)
    img_f = jnp.full((16,), float(_IMG), jnp.float32)
    rowlen = _IMG * _K
    for j in range(rows_per):
        flat = row0 + j
        r = flat - b * _IMG
        rf = jnp.full((16,), r, jnp.int32).astype(jnp.float32)
        ys = 1.0 - 2.0 * (rf + 0.5) / img_f
        pltpu.sync_copy(idx_hbm.at[flat], idx_v)

        def chunk_body(c, carry):
            base = c * 16
            iv = idx_v[pl.ds(base, 16)]
            icl = jnp.maximum(iv, zero16)
            pxg = plsc.load_gather(pts_v, [zero16, icl])
            pyg = plsc.load_gather(pts_v, [one16, icl])
            wcol = lax.shift_right_logical(lane + base, 3)
            xsv = 1.0 - 2.0 * (wcol.astype(jnp.float32) + 0.5) / img_f
            dx = pxg - xsv
            dyv = pyg - ys
            d = dx * dx + dyv * dyv
            dst_v[pl.ds(base, 16)] = jnp.where(iv < 0, -1.0, d)
            return carry

        lax.fori_loop(0, rowlen // 16, chunk_body, jnp.int32(0))
        pltpu.sync_copy(dst_v, dist_hbm.at[flat])


def kernel(points, R, T, focal):
    B, N, _ = points.shape
    img, k = _IMG, _K
    nrows = B * img
    rows_per = nrows // _NSUB
    pts = jnp.transpose(points, (0, 2, 1))
    params = jnp.concatenate(
        [R.reshape(B, 9), T.reshape(B, 3), focal.reshape(B, 1),
         jnp.zeros((B, 3), jnp.float32)], axis=1).reshape(B, 1, 16)

    pxyz = pl.pallas_call(
        _transform_kernel,
        out_shape=jax.ShapeDtypeStruct((B, 3, N), jnp.float32),
        grid=(B,),
        in_specs=[
            pl.BlockSpec((1, 1, 16), lambda b: (b, 0, 0),
                         memory_space=pltpu.SMEM),
            pl.BlockSpec((1, 3, N), lambda b: (b, 0, 0)),
        ],
        out_specs=pl.BlockSpec((1, 3, N), lambda b: (b, 0, 0)),
    )(params, pts)

    mesh = plsc.VectorSubcoreMesh(core_axis_name="c", subcore_axis_name="s")
    cand, counts2d = pl.kernel(
        functools.partial(_sc_bin_kernel, n=N, rows_per=rows_per),
        out_type=[jax.ShapeDtypeStruct((nrows, 4, N), jnp.float32),
                  jax.ShapeDtypeStruct((_NSUB, 16), jnp.int32)],
        mesh=mesh,
        compiler_params=pltpu.CompilerParams(needs_layout_passes=False),
        scratch_types=[pltpu.VMEM((3, N), jnp.float32),
                       pltpu.VMEM((N + 16,), jnp.float32),
                       pltpu.VMEM((N + 16,), jnp.float32),
                       pltpu.VMEM((N + 16,), jnp.float32),
                       pltpu.VMEM((N + 16,), jnp.float32),
                       pltpu.VMEM((16,), jnp.int32)],
    )(pxyz)

    counts_flat = counts2d[:, :rows_per].reshape(nrows)

    out_shapes = (
        jax.ShapeDtypeStruct((B, img, img, k), jnp.int32),
        jax.ShapeDtypeStruct((B, img, img, k), jnp.float32),
    )
    out_spec = pl.BlockSpec((1, 1, img, k),
                            lambda i, counts: (i // img, i % img, 0, 0))
    idx, zbuf = pl.pallas_call(
        functools.partial(_topk_kernel, n=N),
        out_shape=out_shapes,
        grid_spec=pltpu.PrefetchScalarGridSpec(
            num_scalar_prefetch=1,
            grid=(nrows,),
            in_specs=[pl.BlockSpec((1, 4, N), lambda i, counts: (i, 0, 0))],
            out_specs=(out_spec, out_spec),
        ),
    )(counts_flat, cand)

    rowlen = img * k
    dists = pl.kernel(
        functools.partial(_sc_dist_kernel, n=N, rows_per=rows_per),
        out_type=jax.ShapeDtypeStruct((nrows, rowlen), jnp.float32),
        mesh=mesh,
        compiler_params=pltpu.CompilerParams(needs_layout_passes=False),
        scratch_types=[pltpu.VMEM((3, N), jnp.float32),
                       pltpu.VMEM((rowlen,), jnp.int32),
                       pltpu.VMEM((rowlen,), jnp.float32)],
    )(pxyz, idx.reshape(nrows, rowlen))

    return idx, zbuf, dists.reshape(B, img, img, k)
